# baseline (speedup 1.0000x reference)
import jax
import jax.numpy as jnp
from jax.experimental import pallas as pl
from jax.experimental.pallas import tpu as pltpu

N = 4096
D_IN = 256
D_OUT = 256

TILES = (128, 512, 512, 512, 512, 512, 512, 512, 256, 128)
MAXT = max(TILES)
_STARTS = []
_s = 0
for _t in TILES:
    _STARTS.append(_s)
    _s += _t
assert _s == N


def _fused(feat_ref, w_ref, adj_hbm, out_hbm,
           adj_buf, out_buf, support_ref, adj_sem, out_sem):
    n = len(TILES)

    def fetch(i):
        c = pltpu.make_async_copy(
            adj_hbm.at[pl.ds(_STARTS[i], TILES[i]), :],
            adj_buf.at[i % 2, pl.ds(0, TILES[i]), :],
            adj_sem.at[i % 2],
        )
        c.start()
        return c

    fetches = {0: fetch(0), 1: fetch(1)}

    support_ref[:, :] = jnp.dot(
        feat_ref[:, :], w_ref[:, :], preferred_element_type=jnp.float32
    ).astype(jnp.bfloat16)

    writes = {}
    for i in range(n):
        rows = TILES[i]
        fetches[i].wait()
        acc = jnp.dot(
            adj_buf[i % 2, :rows, :].astype(jnp.bfloat16),
            support_ref[:, :],
            preferred_element_type=jnp.float32,
        )
        if i >= 2:
            writes[i - 2].wait()
        out_buf[i % 2, :rows, :] = jnp.maximum(acc, 0.0)
        wc = pltpu.make_async_copy(
            out_buf.at[i % 2, pl.ds(0, rows), :],
            out_hbm.at[pl.ds(_STARTS[i], rows), :],
            out_sem.at[i % 2],
        )
        wc.start()
        writes[i] = wc
        if i + 2 < n:
            fetches[i + 2] = fetch(i + 2)

    writes[n - 2].wait()
    writes[n - 1].wait()


@jax.jit
def kernel(features, adj, weight):
    return pl.pallas_call(
        _fused,
        in_specs=[
            pl.BlockSpec(memory_space=pltpu.VMEM),
            pl.BlockSpec(memory_space=pltpu.VMEM),
            pl.BlockSpec(memory_space=pl.ANY),
        ],
        out_specs=pl.BlockSpec(memory_space=pl.ANY),
        out_shape=jax.ShapeDtypeStruct((N, D_OUT), jnp.float32),
        scratch_shapes=[
            pltpu.VMEM((2, MAXT, N), jnp.float32),
            pltpu.VMEM((2, MAXT, D_OUT), jnp.float32),
            pltpu.VMEM((N, D_OUT), jnp.bfloat16),
            pltpu.SemaphoreType.DMA((2,)),
            pltpu.SemaphoreType.DMA((2,)),
        ],
    )(features, weight, adj)


# speedup vs baseline: 1.1600x; 1.1600x over previous
import jax
import jax.numpy as jnp
from jax.experimental import pallas as pl
from jax.experimental.pallas import tpu as pltpu

N = 4096
D_IN = 256
D_OUT = 256
TM = 512


def _fused(feat_ref, w_ref, adj_ref, out_ref, support_ref):
    i = pl.program_id(0)

    @pl.when(i == 0)
    def _():
        support_ref[:, :] = jnp.dot(
            feat_ref[:, :], w_ref[:, :], preferred_element_type=jnp.float32
        ).astype(jnp.bfloat16)

    out_ref[:, :] = jnp.maximum(
        jnp.dot(
            adj_ref[:, :].astype(jnp.bfloat16),
            support_ref[:, :],
            preferred_element_type=jnp.float32,
        ),
        0.0,
    )


@jax.jit
def kernel(features, adj, weight):
    return pl.pallas_call(
        _fused,
        grid=(N // TM,),
        in_specs=[
            pl.BlockSpec((N, D_IN), lambda i: (0, 0)),
            pl.BlockSpec((D_IN, D_OUT), lambda i: (0, 0)),
            pl.BlockSpec((TM, N), lambda i: (i, 0)),
        ],
        out_specs=pl.BlockSpec((TM, D_OUT), lambda i: (i, 0)),
        out_shape=jax.ShapeDtypeStruct((N, D_OUT), jnp.float32),
        scratch_shapes=[pltpu.VMEM((N, D_OUT), jnp.bfloat16)],
        compiler_params=pltpu.CompilerParams(
            dimension_semantics=("arbitrary",),
        ),
    )(features, weight, adj)
